# baseline (speedup 1.0000x reference)
import functools

import jax
import jax.numpy as jnp
from jax import lax
from jax.experimental import pallas as pl
from jax.experimental.pallas import tpu as pltpu
from jax.experimental.pallas import tpu_sc as plsc

NC = 2
NS = 16
L = 16
CB = 128
BT = 11264
RT = 1024


def _sc_accum_body(mu_hbm, sig_hbm, idx_hbm,
                   sum_mu_hbm, sum_sig_hbm, cnt_hbm,
                   mu_v, sig_v, idx_c, idx_v, row_v, st_v, tab_v, cbuf, pst,
                   acc_mu, acc_sig, acc_cnt, sems,
                   sh_mu, sh_sig, sh_cnt,
                   *, B, C, D, K, BT_):
    CH = C // NC
    cnt_rows = B // NS
    spt = (B - BT_) // NS
    n_chunks = spt // K
    c = lax.axis_index("c")
    s = lax.axis_index("s")
    ch0 = c * CH

    i32 = jnp.int32
    zeros = jnp.zeros((L,), jnp.float32)
    ones = jnp.ones((L,), jnp.float32)
    iota = jax.lax.iota(i32, L)
    row0 = BT_ + s * spt

    def issue(g, b):
        base = row0 + g * K
        pltpu.async_copy(mu_hbm.at[pl.ds(base, K), pl.ds(ch0, CH)],
                         mu_v.at[b], sems.at[b])
        pltpu.async_copy(sig_hbm.at[pl.ds(base, K), pl.ds(ch0, CH)],
                         sig_v.at[b], sems.at[b])

    issue(0, 0)
    issue(1, 1)
    di = [pltpu.async_copy(idx_hbm.at[pl.ds(s * cnt_rows, cnt_rows)], idx_c,
                           sems.at[2]),
          pltpu.async_copy(idx_hbm.at[pl.ds(BT_ + s * spt, spt)], idx_v,
                           sems.at[2])]

    ZU = 8

    def zinit(i, carry):
        for u in range(ZU):
            acc_mu[pl.ds((i * ZU + u) * L, L)] = zeros
            acc_sig[pl.ds((i * ZU + u) * L, L)] = zeros
        return carry
    lax.fori_loop(0, D * CH // (L * ZU), zinit, 0)
    for d in range(D):
        acc_cnt[pl.ds(d * CB, L)] = zeros

    for dsc in di:
        dsc.wait()

    def cstep(q, carry):
        for u in range(4):
            idx_l = idx_c[pl.ds((q * 4 + u) * L, L)]
            plsc.addupdate_scatter(acc_cnt, [idx_l * CB + iota], ones)
        return carry
    lax.fori_loop(0, cnt_rows // (L * 4), cstep, 0)

    def drain(b):
        pltpu.make_async_copy(mu_hbm.at[pl.ds(0, K), pl.ds(0, CH)],
                              mu_v.at[b], sems.at[b]).wait()
        pltpu.make_async_copy(sig_hbm.at[pl.ds(0, K), pl.ds(0, CH)],
                              sig_v.at[b], sems.at[b]).wait()

    G = 8

    def consume(g, b):
        def row(r, carry2):
            dvec = plsc.load_gather(idx_v, [jnp.full((L,), g * K + r, i32)])
            base = dvec * CH
            for j0 in range(0, CH // L, G):
                ms = [mu_v[b, r, pl.ds((j0 + t) * L, L)] for t in range(G)]
                ss = [sig_v[b, r, pl.ds((j0 + t) * L, L)] for t in range(G)]
                ads = [base + (iota + (j0 + t) * L) for t in range(G)]
                for t in range(G):
                    plsc.addupdate_scatter(acc_mu, [ads[t]], ms[t])
                    plsc.addupdate_scatter(acc_sig, [ads[t]], ss[t])
            return carry2
        lax.fori_loop(0, K, row, 0)

    @pl.loop(0, n_chunks, step=2)
    def _chunks(g):
        for b in range(2):
            drain(b)
            consume(g + b, b)

            @pl.when(g + b + 2 < n_chunks)
            def _():
                issue(g + b + 2, b)

    dp = [pltpu.async_copy(acc_mu, sh_mu.at[pl.ds(s * D * CH, D * CH)],
                           sems.at[0]),
          pltpu.async_copy(acc_sig, sh_sig.at[pl.ds(s * D * CH, D * CH)],
                           sems.at[0]),
          pltpu.async_copy(acc_cnt, sh_cnt.at[pl.ds(s * D * CB, D * CB)],
                           sems.at[0])]
    for dsc in dp:
        dsc.wait()
    plsc.subcore_barrier()

    dc = [pltpu.async_copy(sh_cnt.at[pl.ds(t * D * CB + s * CB, L)],
                           st_v.at[pl.ds(t * L, L)], sems.at[1])
          for t in range(NS)]
    for dsc in dc:
        dsc.wait()
    cvec = st_v[pl.ds(0, L)]
    for t in range(1, NS):
        cvec = cvec + st_v[pl.ds(t * L, L)]
    cnt = jnp.sum(cvec)
    csplat = jnp.full((L,), cnt)

    @pl.when(c == 0)
    def _():
        for q in range(C // L):
            cbuf[pl.ds(q * L, L)] = csplat
        pltpu.sync_copy(cbuf, cnt_hbm.at[pl.ds(s * C, C)])

    for arr_sh, arr_out in ((sh_mu, sum_mu_hbm), (sh_sig, sum_sig_hbm)):
        dg = [pltpu.async_copy(arr_sh.at[pl.ds(t * D * CH + s * CH, CH)],
                               pst.at[pl.ds(t * CH, CH)], sems.at[0])
              for t in range(NS)]
        for dsc in dg:
            dsc.wait()
        for j in range(CH // L):
            x = pst[pl.ds(j * L, L)]
            for t in range(1, NS):
                x = x + pst[pl.ds(t * CH + j * L, L)]
            row_v[pl.ds(j * L, L)] = x
        pltpu.sync_copy(row_v, arr_out.at[pl.ds(s * C + ch0, CH)])


def _tc_body(idx_ref, mu_ref, sig_ref, om_ref, os_ref, *, D, R):
    g = pl.program_id(0)
    idx = idx_ref[0, 0, :]
    oh = (lax.broadcasted_iota(jnp.int32, (D, R), 0)
          == idx[None, :]).astype(jnp.float32)
    pm = jnp.dot(oh, mu_ref[...], preferred_element_type=jnp.float32)
    ps = jnp.dot(oh, sig_ref[...], preferred_element_type=jnp.float32)

    @pl.when(g == 0)
    def _():
        om_ref[...] = pm
        os_ref[...] = ps

    @pl.when(g > 0)
    def _():
        om_ref[...] += pm
        os_ref[...] += ps


def _tc_combine_body(cnt_ref, sm_ref, ss_ref, tm_ref, ts_ref,
                     mt_ref, st_ref, om_ref, os_ref):
    cnt = cnt_ref[...]
    present = cnt > 0.0
    recip = 1.0 / jnp.maximum(cnt, 1.0)
    tm = tm_ref[...].reshape(cnt.shape)
    ts = ts_ref[...].reshape(cnt.shape)
    om = jnp.where(present, (sm_ref[...] + tm) * recip, mt_ref[...])
    os = jnp.where(present, (ss_ref[...] + ts) * recip, st_ref[...])
    om_ref[...] = om.reshape(om_ref.shape)
    os_ref[...] = os.reshape(os_ref.shape)


@jax.jit
def _style_stats(mu, sig, mu_table, sig_table, domain_idx):
    B, C = mu.shape
    D = mu_table.shape[0]
    K = 32
    CH = C // NC
    assert D == NS and (B - BT) % (NS * K) == 0 and C % (NC * L) == 0
    assert BT % RT == 0 and B % (NS * L) == 0

    mesh = plsc.VectorSubcoreMesh(core_axis_name="c", subcore_axis_name="s")
    f32 = jnp.float32
    params = pltpu.CompilerParams(needs_layout_passes=False)

    accum = pl.kernel(
        functools.partial(_sc_accum_body, B=B, C=C, D=D, K=K, BT_=BT),
        out_type=(jax.ShapeDtypeStruct((D * C,), f32),
                  jax.ShapeDtypeStruct((D * C,), f32),
                  jax.ShapeDtypeStruct((D * C,), f32)),
        mesh=mesh,
        compiler_params=params,
        scratch_types=[
            pltpu.VMEM((2, K, CH), f32),
            pltpu.VMEM((2, K, CH), f32),
            pltpu.VMEM((B // NS,), jnp.int32),
            pltpu.VMEM(((B - BT) // NS,), jnp.int32),
            pltpu.VMEM((CH,), f32),
            pltpu.VMEM((NS * L,), f32),
            pltpu.VMEM((CH,), f32),
            pltpu.VMEM((C,), f32),
            pltpu.VMEM((NS * CH,), f32),
            pltpu.VMEM((D * CH,), f32),
            pltpu.VMEM((D * CH,), f32),
            pltpu.VMEM((D * CB,), f32),
            pltpu.SemaphoreType.DMA((3,)),
            pltpu.VMEM_SHARED((NS * D * CH,), f32),
            pltpu.VMEM_SHARED((NS * D * CH,), f32),
            pltpu.VMEM_SHARED((NS * D * CB,), f32),
        ],
    )
    sum_mu, sum_sig, cnt = accum(mu, sig, domain_idx)

    idx3 = domain_idx.reshape(B // RT, 1, RT)
    tc = pl.pallas_call(
        functools.partial(_tc_body, D=D, R=RT),
        grid=(BT // RT,),
        in_specs=[
            pl.BlockSpec((1, 1, RT), lambda g: (g, 0, 0)),
            pl.BlockSpec((RT, C), lambda g: (g, 0)),
            pl.BlockSpec((RT, C), lambda g: (g, 0)),
        ],
        out_specs=[
            pl.BlockSpec((D, C), lambda g: (0, 0)),
            pl.BlockSpec((D, C), lambda g: (0, 0)),
        ],
        out_shape=[jax.ShapeDtypeStruct((D, C), f32),
                   jax.ShapeDtypeStruct((D, C), f32)],
        compiler_params=pltpu.CompilerParams(
            dimension_semantics=("arbitrary",)),
    )
    tc_mu, tc_sig = tc(idx3, mu, sig)

    SQ = 128
    combine = pl.pallas_call(
        _tc_combine_body,
        out_shape=[jax.ShapeDtypeStruct((D, C), f32),
                   jax.ShapeDtypeStruct((D, C), f32)],
    )
    return combine(
        cnt.reshape(SQ, -1), sum_mu.reshape(SQ, -1), sum_sig.reshape(SQ, -1),
        tc_mu, tc_sig,
        mu_table.reshape(SQ, -1), sig_table.reshape(SQ, -1))


def kernel(mu, sig, mu_table, sig_table, domain_idx, layer_idx):
    del layer_idx
    return _style_stats(mu, sig, mu_table, sig_table, domain_idx)


# speedup vs baseline: 1.0560x; 1.0560x over previous
import functools

import jax
import jax.numpy as jnp
from jax import lax
from jax.experimental import pallas as pl
from jax.experimental.pallas import tpu as pltpu
from jax.experimental.pallas import tpu_sc as plsc

NC = 2
NS = 16
L = 16
CB = 128
BT = 12288
RT = 2048


def _sc_accum_body(mu_hbm, sig_hbm, idx_hbm,
                   sum_mu_hbm, sum_sig_hbm, cnt_hbm,
                   mu_v, sig_v, idx_c, idx_v, row_v, st_v, tab_v, cbuf, pst,
                   acc_mu, acc_sig, acc_cnt, sems,
                   sh_mu, sh_sig, sh_cnt,
                   *, B, C, D, K, BT_):
    CH = C // NC
    cnt_rows = B // NS
    spt = (B - BT_) // NS
    n_chunks = spt // K
    c = lax.axis_index("c")
    s = lax.axis_index("s")
    ch0 = c * CH

    i32 = jnp.int32
    zeros = jnp.zeros((L,), jnp.float32)
    ones = jnp.ones((L,), jnp.float32)
    iota = jax.lax.iota(i32, L)
    row0 = BT_ + s * spt

    def issue(g, b):
        base = row0 + g * K
        pltpu.async_copy(mu_hbm.at[pl.ds(base, K), pl.ds(ch0, CH)],
                         mu_v.at[b], sems.at[b])
        pltpu.async_copy(sig_hbm.at[pl.ds(base, K), pl.ds(ch0, CH)],
                         sig_v.at[b], sems.at[b])

    issue(0, 0)
    issue(1, 1)
    di = [pltpu.async_copy(idx_hbm.at[pl.ds(s * cnt_rows, cnt_rows)], idx_c,
                           sems.at[2]),
          pltpu.async_copy(idx_hbm.at[pl.ds(BT_ + s * spt, spt)], idx_v,
                           sems.at[2])]

    ZU = 8

    def zinit(i, carry):
        for u in range(ZU):
            acc_mu[pl.ds((i * ZU + u) * L, L)] = zeros
            acc_sig[pl.ds((i * ZU + u) * L, L)] = zeros
        return carry
    lax.fori_loop(0, D * CH // (L * ZU), zinit, 0)
    for d in range(D):
        acc_cnt[pl.ds(d * CB, L)] = zeros

    for dsc in di:
        dsc.wait()

    def cstep(q, carry):
        for u in range(4):
            idx_l = idx_c[pl.ds((q * 4 + u) * L, L)]
            plsc.addupdate_scatter(acc_cnt, [idx_l * CB + iota], ones)
        return carry
    lax.fori_loop(0, cnt_rows // (L * 4), cstep, 0)

    def drain(b):
        pltpu.make_async_copy(mu_hbm.at[pl.ds(0, K), pl.ds(0, CH)],
                              mu_v.at[b], sems.at[b]).wait()
        pltpu.make_async_copy(sig_hbm.at[pl.ds(0, K), pl.ds(0, CH)],
                              sig_v.at[b], sems.at[b]).wait()

    G = 8

    def consume(g, b):
        def row(r, carry2):
            dvec = plsc.load_gather(idx_v, [jnp.full((L,), g * K + r, i32)])
            base = dvec * CH
            for j0 in range(0, CH // L, G):
                ms = [mu_v[b, r, pl.ds((j0 + t) * L, L)] for t in range(G)]
                ss = [sig_v[b, r, pl.ds((j0 + t) * L, L)] for t in range(G)]
                ads = [base + (iota + (j0 + t) * L) for t in range(G)]
                for t in range(G):
                    plsc.addupdate_scatter(acc_mu, [ads[t]], ms[t])
                    plsc.addupdate_scatter(acc_sig, [ads[t]], ss[t])
            return carry2
        lax.fori_loop(0, K, row, 0)

    @pl.loop(0, n_chunks, step=2)
    def _chunks(g):
        for b in range(2):
            drain(b)
            consume(g + b, b)

            @pl.when(g + b + 2 < n_chunks)
            def _():
                issue(g + b + 2, b)

    dp = [pltpu.async_copy(acc_mu, sh_mu.at[pl.ds(s * D * CH, D * CH)],
                           sems.at[0]),
          pltpu.async_copy(acc_sig, sh_sig.at[pl.ds(s * D * CH, D * CH)],
                           sems.at[0]),
          pltpu.async_copy(acc_cnt, sh_cnt.at[pl.ds(s * D * CB, D * CB)],
                           sems.at[0])]
    for dsc in dp:
        dsc.wait()
    plsc.subcore_barrier()

    dc = [pltpu.async_copy(sh_cnt.at[pl.ds(t * D * CB + s * CB, L)],
                           st_v.at[pl.ds(t * L, L)], sems.at[1])
          for t in range(NS)]
    for dsc in dc:
        dsc.wait()
    cvec = st_v[pl.ds(0, L)]
    for t in range(1, NS):
        cvec = cvec + st_v[pl.ds(t * L, L)]
    cnt = jnp.sum(cvec)
    csplat = jnp.full((L,), cnt)

    @pl.when(c == 0)
    def _():
        for q in range(C // L):
            cbuf[pl.ds(q * L, L)] = csplat
        pltpu.sync_copy(cbuf, cnt_hbm.at[pl.ds(s * C, C)])

    for arr_sh, arr_out in ((sh_mu, sum_mu_hbm), (sh_sig, sum_sig_hbm)):
        dg = [pltpu.async_copy(arr_sh.at[pl.ds(t * D * CH + s * CH, CH)],
                               pst.at[pl.ds(t * CH, CH)], sems.at[0])
              for t in range(NS)]
        for dsc in dg:
            dsc.wait()
        for j in range(CH // L):
            x = pst[pl.ds(j * L, L)]
            for t in range(1, NS):
                x = x + pst[pl.ds(t * CH + j * L, L)]
            row_v[pl.ds(j * L, L)] = x
        pltpu.sync_copy(row_v, arr_out.at[pl.ds(s * C + ch0, CH)])


def _tc_body(idx_ref, mu_ref, sig_ref, om_ref, os_ref, *, D, R):
    g = pl.program_id(0)
    idx = idx_ref[0, 0, :]
    oh = (lax.broadcasted_iota(jnp.int32, (D, R), 0)
          == idx[None, :]).astype(jnp.float32)
    pm = jnp.dot(oh, mu_ref[...], preferred_element_type=jnp.float32)
    ps = jnp.dot(oh, sig_ref[...], preferred_element_type=jnp.float32)

    @pl.when(g == 0)
    def _():
        om_ref[...] = pm
        os_ref[...] = ps

    @pl.when(g > 0)
    def _():
        om_ref[...] += pm
        os_ref[...] += ps


def _tc_combine_body(cnt_ref, sm_ref, ss_ref, tm_ref, ts_ref,
                     mt_ref, st_ref, om_ref, os_ref):
    cnt = cnt_ref[...]
    present = cnt > 0.0
    recip = 1.0 / jnp.maximum(cnt, 1.0)
    tm = tm_ref[...].reshape(cnt.shape)
    ts = ts_ref[...].reshape(cnt.shape)
    om = jnp.where(present, (sm_ref[...] + tm) * recip, mt_ref[...])
    os = jnp.where(present, (ss_ref[...] + ts) * recip, st_ref[...])
    om_ref[...] = om.reshape(om_ref.shape)
    os_ref[...] = os.reshape(os_ref.shape)


@jax.jit
def _style_stats(mu, sig, mu_table, sig_table, domain_idx):
    B, C = mu.shape
    D = mu_table.shape[0]
    K = 32
    CH = C // NC
    assert D == NS and (B - BT) % (NS * K) == 0 and C % (NC * L) == 0
    assert BT % RT == 0 and B % (NS * L) == 0

    mesh = plsc.VectorSubcoreMesh(core_axis_name="c", subcore_axis_name="s")
    f32 = jnp.float32
    params = pltpu.CompilerParams(needs_layout_passes=False)

    accum = pl.kernel(
        functools.partial(_sc_accum_body, B=B, C=C, D=D, K=K, BT_=BT),
        out_type=(jax.ShapeDtypeStruct((D * C,), f32),
                  jax.ShapeDtypeStruct((D * C,), f32),
                  jax.ShapeDtypeStruct((D * C,), f32)),
        mesh=mesh,
        compiler_params=params,
        scratch_types=[
            pltpu.VMEM((2, K, CH), f32),
            pltpu.VMEM((2, K, CH), f32),
            pltpu.VMEM((B // NS,), jnp.int32),
            pltpu.VMEM(((B - BT) // NS,), jnp.int32),
            pltpu.VMEM((CH,), f32),
            pltpu.VMEM((NS * L,), f32),
            pltpu.VMEM((CH,), f32),
            pltpu.VMEM((C,), f32),
            pltpu.VMEM((NS * CH,), f32),
            pltpu.VMEM((D * CH,), f32),
            pltpu.VMEM((D * CH,), f32),
            pltpu.VMEM((D * CB,), f32),
            pltpu.SemaphoreType.DMA((3,)),
            pltpu.VMEM_SHARED((NS * D * CH,), f32),
            pltpu.VMEM_SHARED((NS * D * CH,), f32),
            pltpu.VMEM_SHARED((NS * D * CB,), f32),
        ],
    )
    sum_mu, sum_sig, cnt = accum(mu, sig, domain_idx)

    idx3 = domain_idx.reshape(B // RT, 1, RT)
    tc = pl.pallas_call(
        functools.partial(_tc_body, D=D, R=RT),
        grid=(BT // RT,),
        in_specs=[
            pl.BlockSpec((1, 1, RT), lambda g: (g, 0, 0)),
            pl.BlockSpec((RT, C), lambda g: (g, 0)),
            pl.BlockSpec((RT, C), lambda g: (g, 0)),
        ],
        out_specs=[
            pl.BlockSpec((D, C), lambda g: (0, 0)),
            pl.BlockSpec((D, C), lambda g: (0, 0)),
        ],
        out_shape=[jax.ShapeDtypeStruct((D, C), f32),
                   jax.ShapeDtypeStruct((D, C), f32)],
        compiler_params=pltpu.CompilerParams(
            dimension_semantics=("arbitrary",)),
    )
    tc_mu, tc_sig = tc(idx3, mu, sig)

    SQ = 128
    combine = pl.pallas_call(
        _tc_combine_body,
        out_shape=[jax.ShapeDtypeStruct((D, C), f32),
                   jax.ShapeDtypeStruct((D, C), f32)],
    )
    return combine(
        cnt.reshape(SQ, -1), sum_mu.reshape(SQ, -1), sum_sig.reshape(SQ, -1),
        tc_mu, tc_sig,
        mu_table.reshape(SQ, -1), sig_table.reshape(SQ, -1))


def kernel(mu, sig, mu_table, sig_table, domain_idx, layer_idx):
    del layer_idx
    return _style_stats(mu, sig, mu_table, sig_table, domain_idx)
